# baseline (speedup 1.0000x reference)
import functools

import jax
import jax.numpy as jnp
from jax import lax
from jax.experimental import pallas as pl
from jax.experimental.pallas import tpu as pltpu
from jax.experimental.pallas import tpu_sc as plsc

_VOCAB = 50257
_N_POS = 1024
_D = 768
_B = 16
_S = 1024
_N = _B * _S
_NC = 2
_NS = 16
_NW = _NC * _NS
_PER_W = _N // _NW
_C = 64
_NCHUNK = _PER_W // _C
_LANES = _D // 16


def _make_emb_kernel():
  mesh = plsc.VectorSubcoreMesh(core_axis_name="c", subcore_axis_name="s")

  @functools.partial(
      pl.kernel,
      mesh=mesh,
      out_type=jax.ShapeDtypeStruct((_N, _D), jnp.float32),
      scratch_types=[
          pltpu.VMEM((_PER_W,), jnp.int32),
          pltpu.VMEM((_C, _D), jnp.float32),
          pltpu.VMEM((_C, _D), jnp.float32),
          pltpu.SemaphoreType.DMA,
      ],
  )
  def emb(idx_hbm, tok_hbm, pos_hbm, out_hbm, idx_v, tok_v, pos_v, sem):
    wid = lax.axis_index("s") * _NC + lax.axis_index("c")
    base = wid * _PER_W
    s0 = base % _S
    pltpu.sync_copy(idx_hbm.at[pl.ds(base, _PER_W)], idx_v)
    for c in range(_NCHUNK):
      pltpu.async_copy(
          tok_hbm.at[idx_v.at[pl.ds(c * _C, _C)]], tok_v, sem
      ).wait()
      pltpu.sync_copy(pos_hbm.at[pl.ds(s0 + c * _C, _C)], pos_v)

      def add_row(r, _):
        for j in range(_LANES):
          sl = pl.ds(j * 16, 16)
          tok_v[r, sl] = tok_v[r, sl] + pos_v[r, sl]
        return ()

      lax.fori_loop(0, _C, add_row, (), unroll=2)
      pltpu.sync_copy(tok_v, out_hbm.at[pl.ds(base + c * _C, _C)])

  return emb


_emb = _make_emb_kernel()


def kernel(input, token_table, pos_table):
  idx = input.reshape(_N).astype(jnp.int32)
  out = _emb(idx, token_table, pos_table)
  return out.reshape(_B, _S, _D)


# speedup vs baseline: 1.7250x; 1.7250x over previous
import functools

import jax
import jax.numpy as jnp
from jax import lax
from jax.experimental import pallas as pl
from jax.experimental.pallas import tpu as pltpu
from jax.experimental.pallas import tpu_sc as plsc

_VOCAB = 50257
_N_POS = 1024
_D = 768
_B = 16
_S = 1024
_N = _B * _S
_NC = 2
_NS = 16
_NW = _NC * _NS
_CW = _S // _NW
_LANES = _D // 16


def _make_emb_kernel():
  mesh = plsc.VectorSubcoreMesh(core_axis_name="c", subcore_axis_name="s")

  @functools.partial(
      pl.kernel,
      mesh=mesh,
      out_type=jax.ShapeDtypeStruct((_N, _D), jnp.float32),
      scratch_types=[
          pltpu.VMEM((_B, _CW), jnp.int32),
          pltpu.VMEM((_CW, _D), jnp.float32),
          pltpu.VMEM((_CW, _D), jnp.float32),
          pltpu.VMEM((_CW, _D), jnp.float32),
          pltpu.SemaphoreType.DMA,
          pltpu.SemaphoreType.DMA,
          pltpu.SemaphoreType.DMA,
          pltpu.SemaphoreType.DMA,
          pltpu.SemaphoreType.DMA,
          pltpu.SemaphoreType.DMA,
      ],
  )
  def emb(idx_hbm, tok_hbm, pos_hbm, out_hbm,
          idx_v, pos_v, tok0, tok1, semi, semp, sg0, sg1, so0, so1):
    wid = lax.axis_index("s") * _NC + lax.axis_index("c")
    col0 = wid * _CW
    his = [
        pltpu.async_copy(
            idx_hbm.at[pl.ds(b * _S + col0, _CW)], idx_v.at[b], semi)
        for b in range(_B)
    ]
    hp = pltpu.async_copy(pos_hbm.at[pl.ds(col0, _CW)], pos_v, semp)
    toks = [tok0, tok1]
    sgs = [sg0, sg1]
    sos = [so0, so1]
    g = [None, None]
    o = [None, None]
    for h in his:
      h.wait()
    g[0] = pltpu.async_copy(tok_hbm.at[idx_v.at[0]], toks[0], sgs[0])
    hp.wait()
    for b in range(_B):
      cur = b & 1
      nxt = 1 - cur
      if b + 1 < _B:
        if o[nxt] is not None:
          o[nxt].wait()
        g[nxt] = pltpu.async_copy(
            tok_hbm.at[idx_v.at[b + 1]], toks[nxt], sgs[nxt])
      g[cur].wait()
      tok = toks[cur]

      def add_row(r, _, tok=tok):
        for j in range(_LANES):
          sl = pl.ds(j * 16, 16)
          tok[r, sl] = tok[r, sl] + pos_v[r, sl]
        return ()

      lax.fori_loop(0, _CW, add_row, ())
      o[cur] = pltpu.async_copy(
          tok, out_hbm.at[pl.ds(b * _S + col0, _CW)], sos[cur])
    o[0].wait()
    o[1].wait()

  return emb


_emb = _make_emb_kernel()


def kernel(input, token_table, pos_table):
  idx = input.reshape(_N).astype(jnp.int32)
  out = _emb(idx, token_table, pos_table)
  return out.reshape(_B, _S, _D)


# speedup vs baseline: 1.9032x; 1.1033x over previous
import functools

import jax
import jax.numpy as jnp
from jax import lax
from jax.experimental import pallas as pl
from jax.experimental.pallas import tpu as pltpu
from jax.experimental.pallas import tpu_sc as plsc

_VOCAB = 50257
_N_POS = 1024
_D = 768
_B = 16
_S = 1024
_N = _B * _S
_NC = 2
_NS = 16
_NW = _NC * _NS
_CW = _S // _NW
_LANES = _D // 16


def _make_emb_kernel():
  mesh = plsc.VectorSubcoreMesh(core_axis_name="c", subcore_axis_name="s")

  @functools.partial(
      pl.kernel,
      mesh=mesh,
      out_type=jax.ShapeDtypeStruct((_N, _D), jnp.float32),
      scratch_types=[
          pltpu.VMEM((_B, _CW), jnp.int32),
          pltpu.VMEM((_CW, _D), jnp.float32),
          pltpu.VMEM((_CW, _D), jnp.float32),
          pltpu.VMEM((_CW, _D), jnp.float32),
          pltpu.VMEM((_CW, _D), jnp.float32),
          pltpu.SemaphoreType.DMA,
          pltpu.SemaphoreType.DMA,
          pltpu.SemaphoreType.DMA,
          pltpu.SemaphoreType.DMA,
          pltpu.SemaphoreType.DMA,
          pltpu.SemaphoreType.DMA,
          pltpu.SemaphoreType.DMA,
          pltpu.SemaphoreType.DMA,
      ],
  )
  def emb(idx_hbm, tok_hbm, pos_hbm, out_hbm,
          idx_v, pos_v, tok0, tok1, tok2,
          semi, semp, sg0, sg1, sg2, so0, so1, so2):
    wid = lax.axis_index("s") * _NC + lax.axis_index("c")
    col0 = wid * _CW
    his = [
        pltpu.async_copy(
            idx_hbm.at[pl.ds(b * _S + col0, _CW)], idx_v.at[b], semi)
        for b in range(_B)
    ]
    hp = pltpu.async_copy(pos_hbm.at[pl.ds(col0, _CW)], pos_v, semp)
    toks = [tok0, tok1, tok2]
    sgs = [sg0, sg1, sg2]
    sos = [so0, so1, so2]
    nbuf = 3
    g = [None] * nbuf
    o = [None] * nbuf
    for h in his:
      h.wait()
    for c in range(nbuf - 1):
      g[c] = pltpu.async_copy(tok_hbm.at[idx_v.at[c]], toks[c], sgs[c])
    hp.wait()
    for b in range(_B):
      cur = b % nbuf
      g[cur].wait()
      tok = toks[cur]

      def add_row(r, _, tok=tok):
        for j in range(_LANES):
          sl = pl.ds(j * 16, 16)
          tok[r, sl] = tok[r, sl] + pos_v[r, sl]
        return ()

      lax.fori_loop(0, _CW, add_row, ())
      o[cur] = pltpu.async_copy(
          tok, out_hbm.at[pl.ds(b * _S + col0, _CW)], sos[cur])
      c = b + nbuf - 1
      if c < _B:
        k = c % nbuf
        if o[k] is not None:
          o[k].wait()
        g[k] = pltpu.async_copy(tok_hbm.at[idx_v.at[c]], toks[k], sgs[k])
    for k in range(nbuf):
      if o[k] is not None:
        o[k].wait()

  return emb


_emb = _make_emb_kernel()


def kernel(input, token_table, pos_table):
  idx = input.reshape(_N).astype(jnp.int32)
  out = _emb(idx, token_table, pos_table)
  return out.reshape(_B, _S, _D)


# speedup vs baseline: 1.9287x; 1.0134x over previous
import functools

import jax
import jax.numpy as jnp
from jax import lax
from jax.experimental import pallas as pl
from jax.experimental.pallas import tpu as pltpu
from jax.experimental.pallas import tpu_sc as plsc

_VOCAB = 50257
_N_POS = 1024
_D = 768
_B = 16
_S = 1024
_N = _B * _S
_NC = 2
_NS = 16
_NW = _NC * _NS
_CW = _S // _NW
_LANES = _D // 16


def _make_emb_kernel():
  mesh = plsc.VectorSubcoreMesh(core_axis_name="c", subcore_axis_name="s")

  @functools.partial(
      pl.kernel,
      mesh=mesh,
      out_type=jax.ShapeDtypeStruct((_N, _D), jnp.float32),
      scratch_types=[
          pltpu.VMEM((_B, _CW), jnp.int32),
          pltpu.VMEM((_CW, _D), jnp.float32),
          pltpu.VMEM((_CW, _D), jnp.float32),
          pltpu.VMEM((_CW, _D), jnp.float32),
          pltpu.VMEM((_CW, _D), jnp.float32),
          pltpu.VMEM((_CW, _D), jnp.float32),
          pltpu.SemaphoreType.DMA,
          pltpu.SemaphoreType.DMA,
          pltpu.SemaphoreType.DMA,
          pltpu.SemaphoreType.DMA,
          pltpu.SemaphoreType.DMA,
          pltpu.SemaphoreType.DMA,
          pltpu.SemaphoreType.DMA,
          pltpu.SemaphoreType.DMA,
          pltpu.SemaphoreType.DMA,
          pltpu.SemaphoreType.DMA,
      ],
  )
  def emb(idx_hbm, tok_hbm, pos_hbm, out_hbm,
          idx_v, pos_v, tok0, tok1, tok2, tok3,
          semi, semp, sg0, sg1, sg2, sg3, so0, so1, so2, so3):
    wid = lax.axis_index("s") * _NC + lax.axis_index("c")
    col0 = wid * _CW
    his = [
        pltpu.async_copy(
            idx_hbm.at[pl.ds(b * _S + col0, _CW)], idx_v.at[b], semi)
        for b in range(_B)
    ]
    hp = pltpu.async_copy(pos_hbm.at[pl.ds(col0, _CW)], pos_v, semp)
    toks = [tok0, tok1, tok2, tok3]
    sgs = [sg0, sg1, sg2, sg3]
    sos = [so0, so1, so2, so3]
    nbuf = 4
    g = [None] * nbuf
    o = [None] * nbuf
    for h in his:
      h.wait()
    for c in range(nbuf - 1):
      g[c] = pltpu.async_copy(tok_hbm.at[idx_v.at[c]], toks[c], sgs[c])
    hp.wait()
    for b in range(_B):
      cur = b % nbuf
      g[cur].wait()
      tok = toks[cur]

      def add_row(r, _, tok=tok):
        for j in range(_LANES):
          sl = pl.ds(j * 16, 16)
          tok[r, sl] = tok[r, sl] + pos_v[r, sl]
        return ()

      lax.fori_loop(0, _CW, add_row, ())
      o[cur] = pltpu.async_copy(
          tok, out_hbm.at[pl.ds(b * _S + col0, _CW)], sos[cur])
      c = b + nbuf - 1
      if c < _B:
        k = c % nbuf
        if o[k] is not None:
          o[k].wait()
        g[k] = pltpu.async_copy(tok_hbm.at[idx_v.at[c]], toks[k], sgs[k])
    for k in range(nbuf):
      if o[k] is not None:
        o[k].wait()

  return emb


_emb = _make_emb_kernel()


def kernel(input, token_table, pos_table):
  idx = input.reshape(_N).astype(jnp.int32)
  out = _emb(idx, token_table, pos_table)
  return out.reshape(_B, _S, _D)
